# Initial kernel scaffold; baseline (speedup 1.0000x reference)
import jax
import jax.numpy as jnp
from jax.experimental import pallas as pl


def kernel(s, v, edges, r_ij, r_ij_normalized, W1, b1, W2, b2, Wr, br):
    raise NotImplementedError("write your pallas kernel here")


# speedup vs baseline: 5.6523x; 5.6523x over previous
import functools

import jax
import jax.numpy as jnp
from jax import lax
from jax.experimental import pallas as pl
from jax.experimental.pallas import tpu as pltpu
from jax.experimental.pallas import tpu_sc as plsc

N = 10000
E = 320000
EMB = 128
NRBF = 20
RCUT = 5.0

NC = 2
NS = 16
NW = NC * NS
Q = 4
K = EMB // Q
TROW = 6 * K
RROW = 5 * K
AROW = 4 * K
NPAD = 10240
RPT = NPAD // NS
EPW = E // NW
B = 80
NBLK = EPW // B


def _node_pack_body(s_ref, v_ref, w1_ref, b1_ref, w2_ref, b2_ref, out_ref):
    s_blk = s_ref[...]
    h = lax.dot_general(s_blk, w1_ref[...], (((1,), (1,)), ((), ())),
                        preferred_element_type=jnp.float32) + b1_ref[...]
    h = h * (1.0 / (1.0 + jnp.exp(-h)))
    sp = lax.dot_general(h, w2_ref[...], (((1,), (1,)), ((), ())),
                         preferred_element_type=jnp.float32) + b2_ref[...]
    v_blk = v_ref[...]
    for q in range(Q):
        c = q * K
        out_ref[q] = jnp.concatenate(
            [sp[:, c:c + K], sp[:, EMB + c:EMB + c + K],
             sp[:, 2 * EMB + c:2 * EMB + c + K],
             v_blk[:, 0, c:c + K], v_blk[:, 1, c:c + K],
             v_blk[:, 2, c:c + K]], axis=1)


def _rbf_pack_body(r_ref, rh_ref, wr_ref, br_ref, out_ref):
    r = r_ref[...]
    ns = (lax.broadcasted_iota(jnp.int32, (1, NRBF), 1) + 1).astype(jnp.float32)
    rbf = jnp.sin(ns * (jnp.pi / RCUT) * r) / r
    lin = lax.dot_general(rbf, wr_ref[...], (((1,), (1,)), ((), ())),
                          preferred_element_type=jnp.float32) + br_ref[...]
    fc = 0.5 * (jnp.cos((jnp.pi / RCUT) * r) + 1.0)
    fc = fc * (r < RCUT).astype(jnp.float32)
    rp = lin * lin * fc
    rh = rh_ref[...]
    for q in range(Q):
        c = q * K
        r3 = rp[:, 2 * EMB + c:2 * EMB + c + K]
        out_ref[q] = jnp.concatenate(
            [rp[:, c:c + K], rp[:, EMB + c:EMB + c + K],
             rh[:, 0:1] * r3, rh[:, 1:2] * r3, rh[:, 2:3] * r3], axis=1)


def _sc_edge_body(t_hbm, r_hbm, src_hbm, dst_hbm, out_hbm,
                  sidx, didx, rows, rbf, msg, acc, sem):
    cid = lax.axis_index("c")
    sid = lax.axis_index("s")
    wid = sid * NC + cid

    zero = jnp.zeros((16,), jnp.float32)

    def _zero_row(i, carry):
        for l in range(AROW // 16):
            msg[i, pl.ds(l * 16, 16)] = zero
        return carry

    lax.fori_loop(0, B, _zero_row, 0)
    for blk in range(RPT // B):
        pltpu.sync_copy(msg, acc.at[pl.ds(sid * RPT + blk * B, B)])
    plsc.subcore_barrier()

    ebase = wid * EPW

    def _block(i, carry):
        base = ebase + i * B
        pltpu.sync_copy(src_hbm.at[pl.ds(base, B)], sidx)
        pltpu.sync_copy(dst_hbm.at[pl.ds(base, B)], didx)
        pltpu.async_copy(t_hbm.at[sidx], rows, sem).wait()
        pltpu.sync_copy(r_hbm.at[pl.ds(base, B)], rbf)

        def _edge(b, ecarry):
            for l in range(K // 16):
                o = l * 16
                r1 = rbf[b, pl.ds(o, 16)]
                r2 = rbf[b, pl.ds(K + o, 16)]
                sp1 = rows[b, pl.ds(o, 16)]
                sp2 = rows[b, pl.ds(K + o, 16)]
                sp3 = rows[b, pl.ds(2 * K + o, 16)]
                msg[b, pl.ds(o, 16)] = r2 * sp2
                dvv = r1 * sp1
                for d in range(3):
                    rd = rbf[b, pl.ds((2 + d) * K + o, 16)]
                    vd = rows[b, pl.ds((3 + d) * K + o, 16)]
                    msg[b, pl.ds((1 + d) * K + o, 16)] = vd * dvv + rd * sp3
            return ecarry

        lax.fori_loop(0, B, _edge, 0)
        pltpu.sync_copy(msg, acc.at[didx], add=True)
        return carry

    lax.fori_loop(0, NBLK, _block, 0)
    plsc.subcore_barrier()

    for blk in range(RPT // B):
        r0 = sid * RPT + blk * B
        pltpu.sync_copy(acc.at[pl.ds(r0, B)], msg)
        pltpu.sync_copy(msg, out_hbm.at[cid].at[pl.ds(r0, B)])


_sc_edge = functools.partial(
    pl.kernel,
    out_type=jax.ShapeDtypeStruct((NC, NPAD, AROW), jnp.float32),
    mesh=plsc.VectorSubcoreMesh(core_axis_name="c", subcore_axis_name="s",
                                num_cores=NC, num_subcores=NS),
    scratch_types=[
        pltpu.VMEM((B,), jnp.int32),
        pltpu.VMEM((B,), jnp.int32),
        pltpu.VMEM((B, TROW), jnp.float32),
        pltpu.VMEM((B, RROW), jnp.float32),
        pltpu.VMEM((B, AROW), jnp.float32),
        pltpu.VMEM_SHARED((NPAD, AROW), jnp.float32),
        pltpu.SemaphoreType.DMA,
    ],
    compiler_params=pltpu.CompilerParams(use_tc_tiling_on_sc=False),
)(_sc_edge_body)


BN = 1000
BE = 2000


def kernel(s, v, edges, r_ij, r_ij_normalized, W1, b1, W2, b2, Wr, br):
    t_tab = pl.pallas_call(
        _node_pack_body,
        grid=(N // BN,),
        in_specs=[
            pl.BlockSpec((BN, EMB), lambda i: (i, 0)),
            pl.BlockSpec((BN, 3, EMB), lambda i: (i, 0, 0)),
            pl.BlockSpec((EMB, EMB), lambda i: (0, 0)),
            pl.BlockSpec((1, EMB), lambda i: (0, 0)),
            pl.BlockSpec((3 * EMB, EMB), lambda i: (0, 0)),
            pl.BlockSpec((1, 3 * EMB), lambda i: (0, 0)),
        ],
        out_specs=pl.BlockSpec((Q, BN, TROW), lambda i: (0, i, 0)),
        out_shape=jax.ShapeDtypeStruct((Q, N, TROW), jnp.float32),
    )(s, v, W1, b1.reshape(1, EMB), W2, b2.reshape(1, 3 * EMB))

    rh_pad = jnp.pad(r_ij_normalized, ((0, 0), (0, 13)))
    r_tab = pl.pallas_call(
        _rbf_pack_body,
        grid=(E // BE,),
        in_specs=[
            pl.BlockSpec((BE, 1), lambda i: (i, 0)),
            pl.BlockSpec((BE, 16), lambda i: (i, 0)),
            pl.BlockSpec((3 * EMB, NRBF), lambda i: (0, 0)),
            pl.BlockSpec((1, 3 * EMB), lambda i: (0, 0)),
        ],
        out_specs=pl.BlockSpec((Q, BE, RROW), lambda i: (0, i, 0)),
        out_shape=jax.ShapeDtypeStruct((Q, E, RROW), jnp.float32),
    )(r_ij.reshape(E, 1), rh_pad, Wr, br.reshape(1, 3 * EMB))

    dst = edges[:, 0]
    src = edges[:, 1]

    ds_parts, dv_parts = [], []
    for q in range(Q):
        part = _sc_edge(t_tab[q], r_tab[q], src, dst)
        po = (part[0, :N] + part[1, :N])
        ds_parts.append(po[:, :K])
        dv_parts.append(po[:, K:])
    s_out = s + jnp.concatenate(ds_parts, axis=1)
    dv = jnp.stack(
        [jnp.concatenate([p[:, d * K:(d + 1) * K] for p in dv_parts], axis=1)
         for d in range(3)], axis=1)
    v_out = v + dv
    return (s_out, v_out)
